# baseline (speedup 1.0000x reference)
import functools

import jax
import jax.numpy as jnp
from jax import lax
from jax.experimental import pallas as pl
from jax.experimental.pallas import tpu as pltpu
from jax.experimental.pallas import tpu_sc as plsc

D_MODEL = 768
LANES = 16
COLS = D_MODEL // LANES
GROUP_ROWS = 16
ROWS_PER_ITER = 3


def _sc_embed(idx_3d, tok_table, pos_table, batch, seq_len):
    n_tok = batch * seq_len
    info = plsc.get_sparse_core_info()
    nc, ns = info.num_cores, info.num_subcores
    nw = nc * ns
    pos_per_w = seq_len // nw
    n_grp = pos_per_w // GROUP_ROWS

    mesh = plsc.VectorSubcoreMesh(core_axis_name="c", subcore_axis_name="s")

    @functools.partial(
        pl.kernel,
        mesh=mesh,
        out_type=jax.ShapeDtypeStruct((n_tok, D_MODEL), jnp.float32),
        scratch_types=(
            [pltpu.VMEM((batch, pos_per_w), jnp.int32)]
            + [pltpu.VMEM((GROUP_ROWS, D_MODEL), jnp.float32)] * 2
            + [pltpu.VMEM((GROUP_ROWS, D_MODEL), jnp.float32)] * (2 * batch)
            + [pltpu.SemaphoreType.DMA] * (2 + 4 * batch)
        ),
    )
    def k(idx_hbm, tok_hbm, pos_hbm, out_hbm, idx_v, *rest):
        posb = rest[:2]
        bufs = (rest[2:2 + batch], rest[2 + batch:2 + 2 * batch])
        sems = rest[2 + 2 * batch:]
        pisems = sems[:2]
        gsems = (sems[2:2 + batch], sems[2 + batch:2 + 2 * batch])
        osems = (sems[2 + 2 * batch:2 + 3 * batch],
                 sems[2 + 3 * batch:2 + 4 * batch])
        wid = lax.axis_index("s") * nc + lax.axis_index("c")
        pltpu.sync_copy(idx_hbm.at[wid], idx_v)

        posinits = [None] * 2
        gathers = [[None] * batch, [None] * batch]
        outs = [[None] * batch, [None] * batch]

        def issue_group(h):
            g = h % 2
            posinits[g] = pltpu.async_copy(
                pos_hbm.at[pl.ds(wid * pos_per_w + h * GROUP_ROWS,
                                 GROUP_ROWS)],
                posb[g], pisems[g])
            for b in range(batch):
                gathers[g][b] = pltpu.async_copy(
                    tok_hbm.at[idx_v.at[b, pl.ds(h * GROUP_ROWS,
                                                 GROUP_ROWS)]],
                    bufs[g][b], gsems[g][b])

        issue_group(0)
        for h in range(n_grp):
            g = h % 2
            if h + 1 < n_grp:
                if h >= 1:
                    for b in range(batch):
                        outs[g ^ 1][b].wait()
                issue_group(h + 1)
            posinits[g].wait()
            for b in range(batch):
                gathers[g][b].wait()
            my_bufs = bufs[g]
            my_pos = posb[g]

            @plsc.parallel_loop(0, GROUP_ROWS, step=1, unroll=ROWS_PER_ITER)
            def add_body(row, _bufs=my_bufs, _pos=my_pos):
                for cc in range(COLS):
                    sl = pl.ds(cc * LANES, LANES)
                    x = _pos[row, sl]
                    for b in range(batch):
                        plsc.addupdate(_bufs[b].at[row, sl], x)

            for b in range(batch):
                outs[g][b] = pltpu.async_copy(
                    my_bufs[b],
                    out_hbm.at[pl.ds(b * seq_len + wid * pos_per_w
                                     + h * GROUP_ROWS, GROUP_ROWS)],
                    osems[g][b])
        for grp in outs:
            for oc in grp:
                if oc is not None:
                    oc.wait()

    return k(idx_3d, tok_table, pos_table)


def kernel(inputs, tok_table, pos_table):
    b, l = inputs.shape
    nw = 32
    idx_3d = inputs.reshape(b, nw, l // nw).transpose(1, 0, 2)
    out = _sc_embed(idx_3d, tok_table, pos_table, b, l)
    return out.reshape(b, l, D_MODEL)


# speedup vs baseline: 1.0073x; 1.0073x over previous
import functools

import jax
import jax.numpy as jnp
from jax import lax
from jax.experimental import pallas as pl
from jax.experimental.pallas import tpu as pltpu
from jax.experimental.pallas import tpu_sc as plsc

D_MODEL = 768
LANES = 16
COLS = D_MODEL // LANES
GROUP_ROWS = 16
ROWS_PER_ITER = 2


def _sc_embed(idx_3d, tok_table, pos_table, batch, seq_len):
    n_tok = batch * seq_len
    info = plsc.get_sparse_core_info()
    nc, ns = info.num_cores, info.num_subcores
    nw = nc * ns
    pos_per_w = seq_len // nw
    n_grp = pos_per_w // GROUP_ROWS

    mesh = plsc.VectorSubcoreMesh(core_axis_name="c", subcore_axis_name="s")

    @functools.partial(
        pl.kernel,
        mesh=mesh,
        out_type=jax.ShapeDtypeStruct((n_tok, D_MODEL), jnp.float32),
        scratch_types=(
            [pltpu.VMEM((batch, pos_per_w), jnp.int32)]
            + [pltpu.VMEM((GROUP_ROWS, D_MODEL), jnp.float32)] * 2
            + [pltpu.VMEM((GROUP_ROWS, D_MODEL), jnp.float32)] * (2 * batch)
            + [pltpu.SemaphoreType.DMA] * (2 + 4 * batch)
        ),
    )
    def k(idx_hbm, tok_hbm, pos_hbm, out_hbm, idx_v, *rest):
        posb = rest[:2]
        bufs = (rest[2:2 + batch], rest[2 + batch:2 + 2 * batch])
        sems = rest[2 + 2 * batch:]
        pisems = sems[:2]
        gsems = (sems[2:2 + batch], sems[2 + batch:2 + 2 * batch])
        osems = (sems[2 + 2 * batch:2 + 3 * batch],
                 sems[2 + 3 * batch:2 + 4 * batch])
        wid = lax.axis_index("s") * nc + lax.axis_index("c")
        pltpu.sync_copy(idx_hbm.at[wid], idx_v)

        posinits = [None] * 2
        gathers = [[None] * batch, [None] * batch]
        outs = [[None] * batch, [None] * batch]

        def issue_group(h):
            g = h % 2
            posinits[g] = pltpu.async_copy(
                pos_hbm.at[pl.ds(wid * pos_per_w + h * GROUP_ROWS,
                                 GROUP_ROWS)],
                posb[g], pisems[g])
            for b in range(batch):
                gathers[g][b] = pltpu.async_copy(
                    tok_hbm.at[idx_v.at[b, pl.ds(h * GROUP_ROWS,
                                                 GROUP_ROWS)]],
                    bufs[g][b], gsems[g][b])

        issue_group(0)
        for h in range(n_grp):
            g = h % 2
            if h + 1 < n_grp:
                if h >= 1:
                    for b in range(batch):
                        outs[g ^ 1][b].wait()
                issue_group(h + 1)
            posinits[g].wait()
            for b in range(batch):
                gathers[g][b].wait()
            my_bufs = bufs[g]
            my_pos = posb[g]

            @plsc.parallel_loop(0, GROUP_ROWS, step=1, unroll=ROWS_PER_ITER)
            def add_body(row, _bufs=my_bufs, _pos=my_pos):
                for cc in range(COLS):
                    sl = pl.ds(cc * LANES, LANES)
                    x = _pos[row, sl]
                    for b in range(batch):
                        plsc.addupdate(_bufs[b].at[row, sl], x)

            for b in range(batch):
                outs[g][b] = pltpu.async_copy(
                    my_bufs[b],
                    out_hbm.at[pl.ds(b * seq_len + wid * pos_per_w
                                     + h * GROUP_ROWS, GROUP_ROWS)],
                    osems[g][b])
        for grp in outs:
            for oc in grp:
                if oc is not None:
                    oc.wait()

    return k(idx_3d, tok_table, pos_table)


def kernel(inputs, tok_table, pos_table):
    b, l = inputs.shape
    nw = 32
    idx_3d = inputs.reshape(b, nw, l // nw).transpose(1, 0, 2)
    out = _sc_embed(idx_3d, tok_table, pos_table, b, l)
    return out.reshape(b, l, D_MODEL)
